# baseline (speedup 1.0000x reference)
import math
import functools
import jax
import jax.numpy as jnp
from jax.experimental import pallas as pl

N = 10000
M = 10000
EI = 160000
D = 256
L = 4
BLK = 200


def _proj_body(x_ref, w_ref, o_ref):
    o_ref[...] = jnp.dot(x_ref[...], w_ref[...],
                         preferred_element_type=jnp.float32)


def _proj(x, w_pad):
    R = x.shape[0]
    return pl.pallas_call(
        _proj_body,
        grid=(R // BLK,),
        in_specs=[
            pl.BlockSpec((BLK, D), lambda i: (i, jnp.int32(0))),
            pl.BlockSpec((D, 8), lambda i: (jnp.int32(0), jnp.int32(0))),
        ],
        out_specs=pl.BlockSpec((BLK, 8), lambda i: (i, jnp.int32(0))),
        out_shape=jax.ShapeDtypeStruct((R, 8), jnp.float32),
    )(x, w_pad)


def _input_body(x_ref, w_ref, b_ref, o_ref):
    o_ref[...] = jax.nn.relu(
        jnp.dot(x_ref[...], w_ref[...], preferred_element_type=jnp.float32)
        + b_ref[...])


def _input_proj(x, W0, b0):
    return pl.pallas_call(
        _input_body,
        grid=(N // BLK,),
        in_specs=[
            pl.BlockSpec((BLK, D), lambda i: (i, jnp.int32(0))),
            pl.BlockSpec((D, D), lambda i: (jnp.int32(0), jnp.int32(0))),
            pl.BlockSpec((1, D), lambda i: (jnp.int32(0), jnp.int32(0))),
        ],
        out_specs=pl.BlockSpec((BLK, D), lambda i: (i, jnp.int32(0))),
        out_shape=jax.ShapeDtypeStruct((N, D), jnp.float32),
    )(x, W0, b0.reshape(1, D))


def _update_body(agg1_ref, agg2_ref, x0_ref, w_ref, o_ref, *, alpha, beta,
                 relu):
    xn = (agg1_ref[...] + agg2_ref[...]) * 0.5
    xi = (1.0 - alpha) * xn + alpha * x0_ref[...]
    o = (1.0 - beta) * xi + beta * jnp.dot(
        xi, w_ref[...], preferred_element_type=jnp.float32)
    if relu:
        o = jax.nn.relu(o)
    o_ref[...] = o


def _update(agg1, agg2, x0, W, alpha, beta, relu):
    body = functools.partial(_update_body, alpha=alpha, beta=beta, relu=relu)
    return pl.pallas_call(
        body,
        grid=(N // BLK,),
        in_specs=[
            pl.BlockSpec((BLK, D), lambda i: (i, jnp.int32(0))),
            pl.BlockSpec((BLK, D), lambda i: (i, jnp.int32(0))),
            pl.BlockSpec((BLK, D), lambda i: (i, jnp.int32(0))),
            pl.BlockSpec((D, D), lambda i: (jnp.int32(0), jnp.int32(0))),
        ],
        out_specs=pl.BlockSpec((BLK, D), lambda i: (i, jnp.int32(0))),
        out_shape=jax.ShapeDtypeStruct((N, D), jnp.float32),
    )(agg1, agg2, x0, W)


def kernel(x, e, hyperedge_index, W0, b0, Wn, We, Wa, ba, wdw, wdb, wgw, wgb,
           dist_v2e, dist_e2v, deg_v2e, deg_e2v):
    row = hyperedge_index[0].astype(jnp.int32)
    col = hyperedge_index[1].astype(jnp.int32)

    ones = jnp.ones((EI,), jnp.float32)
    dV = jax.ops.segment_sum(ones, row, N)
    cnt = jax.ops.segment_sum(ones, col, M)
    dE = jax.ops.segment_sum(dV[row], col, M) / jnp.maximum(cnt, 1.0)
    degV = jnp.where(dV > 0, dV ** -0.5, 1.0)[:, None].astype(jnp.float32)
    degE = jnp.where(dE > 0, dE ** -0.5, 1.0)[:, None].astype(jnp.float32)

    x = _input_proj(x, W0, b0)
    x0 = x
    e0 = e
    lamda, alpha = 0.5, 0.1

    for i in range(L):
        beta = math.log(lamda / (i + 1) + 1)
        wx = jnp.stack([Wa[i, :D], Wa[i, D:]], axis=1)
        we = jnp.stack([Wa[i, D:], Wa[i, :D]], axis=1)
        wx_pad = jnp.pad(wx, ((0, 0), (0, 6)))
        we_pad = jnp.pad(we, ((0, 0), (0, 6)))
        xp = _proj(x, wx_pad)
        ep = _proj(e, we_pad)

        add1 = dist_v2e * wdw[i] + wdb[i] + deg_v2e * wgw[i] + wgb[i] + ba[i]
        add2 = dist_e2v * wdw[i] + wdb[i] + deg_e2v * wgw[i] + wgb[i] + ba[i]

        s1 = xp[row, 0] + ep[col, 0] + add1
        s2 = ep[col, 1] + xp[row, 1] + add2

        p1 = jnp.exp(s1)
        a1 = p1 / jax.ops.segment_sum(p1, col, M)[col]
        p2 = jnp.exp(s2)
        a2 = p2 / jax.ops.segment_sum(p2, row, N)[row]

        Xe1 = jax.ops.segment_sum(x[row] * a1[:, None], col, M) * degE
        Xv1 = jax.ops.segment_sum(Xe1[col] * a2[:, None], row, N) * degV
        Xv2 = jax.ops.segment_sum(e[col] * a2[:, None], row, N) * degV
        Xe2 = jax.ops.segment_sum(Xv2[row] * a1[:, None], col, M) * degE

        relu = i < L - 1
        x = _update(Xv1, Xv2, x0, Wn[i], alpha, beta, relu)
        e = _update(Xe1, Xe2, e0, We[i], alpha, beta, relu)

    return (x, e)


# speedup vs baseline: 3.6510x; 3.6510x over previous
import math
import functools
import jax
import jax.numpy as jnp
from jax.experimental import pallas as pl

N = 10000
M = 10000
EI = 160000
D = 256
L = 4
BLK = 200


def _proj_body(x_ref, w_ref, o_ref):
    o_ref[...] = jnp.dot(x_ref[...], w_ref[...],
                         preferred_element_type=jnp.float32)


def _proj(x, w_pad):
    R = x.shape[0]
    return pl.pallas_call(
        _proj_body,
        grid=(R // BLK,),
        in_specs=[
            pl.BlockSpec((BLK, D), lambda i: (i, jnp.int32(0))),
            pl.BlockSpec((D, 128), lambda i: (jnp.int32(0), jnp.int32(0))),
        ],
        out_specs=pl.BlockSpec((BLK, 128), lambda i: (i, jnp.int32(0))),
        out_shape=jax.ShapeDtypeStruct((R, 128), jnp.float32),
    )(x, w_pad)


def _input_body(x_ref, w_ref, b_ref, o_ref):
    o_ref[...] = jax.nn.relu(
        jnp.dot(x_ref[...], w_ref[...], preferred_element_type=jnp.float32)
        + b_ref[...])


def _input_proj(x, W0, b0):
    return pl.pallas_call(
        _input_body,
        grid=(N // BLK,),
        in_specs=[
            pl.BlockSpec((BLK, D), lambda i: (i, jnp.int32(0))),
            pl.BlockSpec((D, D), lambda i: (jnp.int32(0), jnp.int32(0))),
            pl.BlockSpec((1, D), lambda i: (jnp.int32(0), jnp.int32(0))),
        ],
        out_specs=pl.BlockSpec((BLK, D), lambda i: (i, jnp.int32(0))),
        out_shape=jax.ShapeDtypeStruct((N, D), jnp.float32),
    )(x, W0, b0.reshape(1, D))


def _update_body(agg1_ref, agg2_ref, x0_ref, w_ref, o_ref, *, alpha, beta,
                 relu):
    xn = (agg1_ref[...] + agg2_ref[...]) * 0.5
    xi = (1.0 - alpha) * xn + alpha * x0_ref[...]
    o = (1.0 - beta) * xi + beta * jnp.dot(
        xi, w_ref[...], preferred_element_type=jnp.float32)
    if relu:
        o = jax.nn.relu(o)
    o_ref[...] = o


def _update(agg1, agg2, x0, W, alpha, beta, relu):
    body = functools.partial(_update_body, alpha=alpha, beta=beta, relu=relu)
    return pl.pallas_call(
        body,
        grid=(N // BLK,),
        in_specs=[
            pl.BlockSpec((BLK, D), lambda i: (i, jnp.int32(0))),
            pl.BlockSpec((BLK, D), lambda i: (i, jnp.int32(0))),
            pl.BlockSpec((BLK, D), lambda i: (i, jnp.int32(0))),
            pl.BlockSpec((D, D), lambda i: (jnp.int32(0), jnp.int32(0))),
        ],
        out_specs=pl.BlockSpec((BLK, D), lambda i: (i, jnp.int32(0))),
        out_shape=jax.ShapeDtypeStruct((N, D), jnp.float32),
    )(agg1, agg2, x0, W)


def kernel(x, e, hyperedge_index, W0, b0, Wn, We, Wa, ba, wdw, wdb, wgw, wgb,
           dist_v2e, dist_e2v, deg_v2e, deg_e2v):
    row = hyperedge_index[0].astype(jnp.int32)
    col = hyperedge_index[1].astype(jnp.int32)

    ones = jnp.ones((EI,), jnp.float32)
    dV = jax.ops.segment_sum(ones, row, N)
    cnt = jax.ops.segment_sum(ones, col, M)
    dE = jax.ops.segment_sum(dV[row], col, M) / jnp.maximum(cnt, 1.0)
    degV = jnp.where(dV > 0, dV ** -0.5, 1.0)[:, None].astype(jnp.float32)
    degE = jnp.where(dE > 0, dE ** -0.5, 1.0)[:, None].astype(jnp.float32)

    x = _input_proj(x, W0, b0)
    x0 = x
    e0 = e
    lamda, alpha = 0.5, 0.1

    for i in range(L):
        beta = math.log(lamda / (i + 1) + 1)
        wx = jnp.stack([Wa[i, :D], Wa[i, D:]], axis=1)
        we = jnp.stack([Wa[i, D:], Wa[i, :D]], axis=1)
        wx_pad = jnp.pad(wx, ((0, 0), (0, 126)))
        we_pad = jnp.pad(we, ((0, 0), (0, 126)))
        xp = _proj(x, wx_pad)
        ep = _proj(e, we_pad)

        add1 = dist_v2e * wdw[i] + wdb[i] + deg_v2e * wgw[i] + wgb[i] + ba[i]
        add2 = dist_e2v * wdw[i] + wdb[i] + deg_e2v * wgw[i] + wgb[i] + ba[i]

        s1 = xp[row, 0] + ep[col, 0] + add1
        s2 = ep[col, 1] + xp[row, 1] + add2

        p1 = jnp.exp(s1)
        a1 = p1 / jax.ops.segment_sum(p1, col, M)[col]
        p2 = jnp.exp(s2)
        a2 = p2 / jax.ops.segment_sum(p2, row, N)[row]

        Xe1 = jax.ops.segment_sum(x[row] * a1[:, None], col, M) * degE
        Xv1 = jax.ops.segment_sum(Xe1[col] * a2[:, None], row, N) * degV
        Xv2 = jax.ops.segment_sum(e[col] * a2[:, None], row, N) * degV
        Xe2 = jax.ops.segment_sum(Xv2[row] * a1[:, None], col, M) * degE

        relu = i < L - 1
        x = _update(Xv1, Xv2, x0, Wn[i], alpha, beta, relu)
        e = _update(Xe1, Xe2, e0, We[i], alpha, beta, relu)

    return (x, e)
